# Initial kernel scaffold; baseline (speedup 1.0000x reference)
import jax
import jax.numpy as jnp
from jax.experimental import pallas as pl


def kernel(x, s0_idx, s0_val, s1_idx, s1_val, s2_idx, s2_val, W00, W01, W10, W11, W12):
    raise NotImplementedError("write your pallas kernel here")


# speedup vs baseline: 1.4101x; 1.4101x over previous
import functools

import jax
import jax.numpy as jnp
from jax import lax
from jax.experimental import pallas as pl
from jax.experimental.pallas import tpu as pltpu
from jax.experimental.pallas import tpu_sc as plsc

N = 10000
NPAD = 10240
D = 256
DC = 128
E = 160000
NC = 2
NS = 16
B = 128
EPAD = 163840
TILE_E = EPAD // (NC * NS)
NBATCH = TILE_E // B

TASKS = [
    [(0, 0), (1, 1)],
    [(2, 0), (3, 1)],
    [(4, 0), (5, 1), (6, 2)],
    [(7, 0), (8, 1), (9, 2)],
]


def _mm_body(x_ref, w_ref, o_ref):
    o_ref[0] = jnp.dot(x_ref[...], w_ref[0], preferred_element_type=jnp.float32)


def _make_tables(x, wstack):
    BN = 2000
    return pl.pallas_call(
        _mm_body,
        grid=(N // BN, 10),
        in_specs=[
            pl.BlockSpec((BN, D), lambda i, t: (i, 0)),
            pl.BlockSpec((1, D, DC), lambda i, t: (t, 0, 0)),
        ],
        out_specs=pl.BlockSpec((1, BN, DC), lambda i, t: (t, i, 0)),
        out_shape=jax.ShapeDtypeStruct((10, N, DC), jnp.float32),
    )(x, wstack)


def _sc_body(tabs, src0, src1, src2, dst0, dst1, dst2, val0, val1, val2,
             out, acc, idx_b, dst_b, val_b, rows, zbuf, sem):
    cid = lax.axis_index("c")
    sid = lax.axis_index("s")
    srcs = (src0, src1, src2)
    dsts = (dst0, dst1, dst2)
    vals = (val0, val1, val2)

    def _zb(r, _):
        for j in range(8):
            zbuf[r, pl.ds(j * 16, 16)] = jnp.zeros((16,), jnp.float32)
        return 0
    lax.fori_loop(0, 64, _zb, 0)

    for c in range(4):
        for k in range(10):
            pltpu.sync_copy(zbuf, acc.at[pl.ds(sid * 640 + k * 64, 64), :])
        plsc.subcore_barrier()

        for (t, s) in TASKS[c]:
            sref, dref, vref = srcs[s], dsts[s], vals[s]
            base = cid * (EPAD // NC) + sid * TILE_E

            def _batch(b, _, t=t, sref=sref, dref=dref, vref=vref, base=base):
                off = base + b * B
                pltpu.sync_copy(sref.at[pl.ds(off, B)], idx_b)
                pltpu.sync_copy(dref.at[pl.ds(off, B)], dst_b)
                pltpu.sync_copy(vref.at[pl.ds(off, B)], val_b)
                for j in range(8):
                    idx_b[pl.ds(j * 16, 16)] = idx_b[pl.ds(j * 16, 16)] + t * N
                pltpu.async_copy(tabs.at[idx_b], rows, sem).wait()

                def _scale(g, _):
                    vv = val_b[pl.ds(g * 16, 16)]
                    for l in range(16):
                        v = vv[l]
                        e = g * 16 + l
                        for j in range(8):
                            sl = pl.ds(j * 16, 16)
                            rows[e, sl] = rows[e, sl] * v
                    return 0
                lax.fori_loop(0, B // 16, _scale, 0)
                pltpu.sync_copy(rows, acc.at[dst_b], add=True)
                return 0

            lax.fori_loop(0, NBATCH, _batch, 0)

        plsc.subcore_barrier()
        pltpu.sync_copy(acc.at[pl.ds(sid * 640, 640), :],
                        out.at[cid, c, pl.ds(sid * 640, 640), :])


def _sc_spmm(tabs, srcs, dsts, vals):
    mesh = plsc.VectorSubcoreMesh(core_axis_name="c", subcore_axis_name="s")
    kern = functools.partial(
        pl.kernel,
        mesh=mesh,
        out_type=jax.ShapeDtypeStruct((NC, 4, NPAD, DC), jnp.float32),
        scratch_types=[
            pltpu.VMEM_SHARED((NPAD, DC), jnp.float32),
            pltpu.VMEM((B,), jnp.int32),
            pltpu.VMEM((B,), jnp.int32),
            pltpu.VMEM((B,), jnp.float32),
            pltpu.VMEM((B, DC), jnp.float32),
            pltpu.VMEM((64, DC), jnp.float32),
            pltpu.SemaphoreType.DMA,
        ],
    )(_sc_body)
    return kern(tabs, *srcs, *dsts, *vals)


def _ep_body(p_ref, x_ref, o_ref):
    c0 = p_ref[0, 0] + p_ref[1, 0]
    c1 = p_ref[0, 1] + p_ref[1, 1]
    c2 = p_ref[0, 2] + p_ref[1, 2]
    c3 = p_ref[0, 3] + p_ref[1, 3]
    o_ref[:, 0:DC] = jnp.maximum(jnp.maximum(c0, c2), 0.0)
    o_ref[:, DC:2 * DC] = jnp.maximum(jnp.maximum(c1, c3), 0.0)
    o_ref[:, 2 * DC:] = x_ref[...]


def _epilogue(part, x):
    BN = 2000
    return pl.pallas_call(
        _ep_body,
        grid=(N // BN,),
        in_specs=[
            pl.BlockSpec((NC, 4, BN, DC), lambda i: (0, 0, i, 0)),
            pl.BlockSpec((BN, D), lambda i: (i, 0)),
        ],
        out_specs=pl.BlockSpec((BN, 2 * D), lambda i: (i, 0)),
        out_shape=jax.ShapeDtypeStruct((N, 2 * D), jnp.float32),
    )(part, x)


def kernel(x, s0_idx, s0_val, s1_idx, s1_val, s2_idx, s2_val,
           W00, W01, W10, W11, W12):
    x = x.astype(jnp.float32)
    wstack = jnp.stack([
        W00[:, :DC], W01[:, :DC],
        W00[:, DC:], W01[:, DC:],
        W10[:, :DC], W11[:, :DC], W12[:, :DC],
        W10[:, DC:], W11[:, DC:], W12[:, DC:],
    ])
    tabs = _make_tables(x, wstack).reshape(10 * N, DC)

    pad = EPAD - E
    srcs, dsts, vals = [], [], []
    for idx, val in ((s0_idx, s0_val), (s1_idx, s1_val), (s2_idx, s2_val)):
        idx = idx.astype(jnp.int32)
        srcs.append(jnp.pad(idx[1], (0, pad)))
        dsts.append(jnp.pad(idx[0], (0, pad)))
        vals.append(jnp.pad(val.astype(jnp.float32), (0, pad)))

    part = _sc_spmm(tabs, srcs, dsts, vals)
    return _epilogue(part, x)


# speedup vs baseline: 1.8175x; 1.2889x over previous
import functools

import jax
import jax.numpy as jnp
from jax import lax
from jax.experimental import pallas as pl
from jax.experimental.pallas import tpu as pltpu
from jax.experimental.pallas import tpu_sc as plsc

N = 10000
NPAD = 10112
RPT = 632
D = 256
DC = 128
E = 160000
NC = 2
NS = 16
B = 128
EPAD = 163840
NBTOT = EPAD // B
TILE_E = EPAD // (NC * NS)
NBATCH = TILE_E // B

TASKS = [
    [(0, 0), (1, 1)],
    [(2, 0), (3, 1)],
    [(4, 0), (5, 1), (6, 2)],
    [(7, 0), (8, 1), (9, 2)],
]


def _mm_body(x_ref, w_ref, o_ref):
    o_ref[0] = jnp.dot(x_ref[...], w_ref[0], preferred_element_type=jnp.float32)


def _make_tables(x, wstack):
    BN = 2000
    return pl.pallas_call(
        _mm_body,
        grid=(N // BN, 10),
        in_specs=[
            pl.BlockSpec((BN, D), lambda i, t: (i, 0)),
            pl.BlockSpec((1, D, DC), lambda i, t: (t, 0, 0)),
        ],
        out_specs=pl.BlockSpec((1, BN, DC), lambda i, t: (t, i, 0)),
        out_shape=jax.ShapeDtypeStruct((10, N, DC), jnp.float32),
    )(x, wstack)


def _sc_body(tabs, srcr, dstr, valr, zer,
             out, acc, sbuf, dbuf, vbuf, idx2, rows2, gsem, ssem):
    cid = lax.axis_index("c")
    sid = lax.axis_index("s")
    rowbase = cid * (NBTOT // NC) + sid * NBATCH

    for c in range(4):
        pltpu.sync_copy(zer, acc.at[pl.ds(sid * RPT, RPT), :])
        plsc.subcore_barrier()

        for (t, s) in TASKS[c]:
            pltpu.sync_copy(srcr.at[s, pl.ds(rowbase, NBATCH), :], sbuf)
            pltpu.sync_copy(dstr.at[s, pl.ds(rowbase, NBATCH), :], dbuf)
            pltpu.sync_copy(valr.at[s, pl.ds(rowbase, NBATCH), :], vbuf)

            def _prep(b, bi, t=t):
                for j in range(8):
                    sl = pl.ds(j * 16, 16)
                    idx2[bi, sl] = sbuf[b, sl] + t * N

            def _gather(bi):
                pltpu.async_copy(tabs.at[idx2.at[bi]], rows2.at[bi],
                                 gsem.at[bi])

            def _gwait(bi):
                pltpu.make_async_copy(tabs.at[idx2.at[bi]], rows2.at[bi],
                                      gsem.at[bi]).wait()

            def _swait(bi):
                pltpu.make_async_copy(rows2.at[bi], acc.at[dbuf.at[0]],
                                      ssem.at[bi]).wait()

            _prep(0, 0)
            _gather(0)

            def _batch(b, _):
                bi = b % 2
                bo = (b + 1) % 2

                @pl.when(b + 1 < NBATCH)
                def _():
                    @pl.when(b >= 1)
                    def _():
                        _swait(bo)
                    _prep(b + 1, bo)
                    _gather(bo)

                _gwait(bi)

                def _scale(g, _):
                    vv = vbuf[b, pl.ds(g * 16, 16)]
                    for l in range(16):
                        v = vv[l]
                        e = g * 16 + l
                        for j in range(8):
                            sl = pl.ds(j * 16, 16)
                            rows2[bi, e, sl] = rows2[bi, e, sl] * v
                    return 0
                lax.fori_loop(0, 8, _scale, 0)

                pltpu.async_copy(rows2.at[bi], acc.at[dbuf.at[b]],
                                 ssem.at[bi], add=True)
                return 0

            lax.fori_loop(0, NBATCH, _batch, 0)
            _swait(0)
            _swait(1)

        plsc.subcore_barrier()
        pltpu.sync_copy(acc.at[pl.ds(sid * RPT, RPT), :],
                        out.at[cid, c, pl.ds(sid * RPT, RPT), :])


def _sc_spmm(tabs, srcr, dstr, valr, zer):
    mesh = plsc.VectorSubcoreMesh(core_axis_name="c", subcore_axis_name="s")
    kern = functools.partial(
        pl.kernel,
        mesh=mesh,
        out_type=jax.ShapeDtypeStruct((NC, 4, NPAD, DC), jnp.float32),
        scratch_types=[
            pltpu.VMEM_SHARED((NPAD, DC), jnp.float32),
            pltpu.VMEM((NBATCH, B), jnp.int32),
            pltpu.VMEM((NBATCH, B), jnp.int32),
            pltpu.VMEM((NBATCH, B), jnp.float32),
            pltpu.VMEM((2, B), jnp.int32),
            pltpu.VMEM((2, B, DC), jnp.float32),
            pltpu.SemaphoreType.DMA((2,)),
            pltpu.SemaphoreType.DMA((2,)),
        ],
    )(_sc_body)
    return kern(tabs, srcr, dstr, valr, zer)


def _ep_body(p_ref, x_ref, o_ref):
    c0 = p_ref[0, 0] + p_ref[1, 0]
    c1 = p_ref[0, 1] + p_ref[1, 1]
    c2 = p_ref[0, 2] + p_ref[1, 2]
    c3 = p_ref[0, 3] + p_ref[1, 3]
    o_ref[:, 0:DC] = jnp.maximum(jnp.maximum(c0, c2), 0.0)
    o_ref[:, DC:2 * DC] = jnp.maximum(jnp.maximum(c1, c3), 0.0)
    o_ref[:, 2 * DC:] = x_ref[...]


def _epilogue(part, x):
    BN = 2000
    return pl.pallas_call(
        _ep_body,
        grid=(N // BN,),
        in_specs=[
            pl.BlockSpec((NC, 4, BN, DC), lambda i: (0, 0, i, 0)),
            pl.BlockSpec((BN, D), lambda i: (i, 0)),
        ],
        out_specs=pl.BlockSpec((BN, 2 * D), lambda i: (i, 0)),
        out_shape=jax.ShapeDtypeStruct((N, 2 * D), jnp.float32),
    )(part, x)


def kernel(x, s0_idx, s0_val, s1_idx, s1_val, s2_idx, s2_val,
           W00, W01, W10, W11, W12):
    x = x.astype(jnp.float32)
    wstack = jnp.stack([
        W00[:, :DC], W01[:, :DC],
        W00[:, DC:], W01[:, DC:],
        W10[:, :DC], W11[:, :DC], W12[:, :DC],
        W10[:, DC:], W11[:, DC:], W12[:, DC:],
    ])
    tabs = _make_tables(x, wstack).reshape(10 * N, DC)

    pad = EPAD - E
    srcs, dsts, vals = [], [], []
    for idx, val in ((s0_idx, s0_val), (s1_idx, s1_val), (s2_idx, s2_val)):
        idx = idx.astype(jnp.int32)
        srcs.append(jnp.pad(idx[1], (0, pad)).reshape(NBTOT, B))
        dsts.append(jnp.pad(idx[0], (0, pad)).reshape(NBTOT, B))
        vals.append(jnp.pad(val.astype(jnp.float32), (0, pad)).reshape(NBTOT, B))

    zer = jnp.zeros((RPT, DC), jnp.float32)
    part = _sc_spmm(tabs, jnp.stack(srcs), jnp.stack(dsts), jnp.stack(vals),
                    zer)
    return _epilogue(part, x)


# speedup vs baseline: 2.1726x; 1.1954x over previous
import functools

import jax
import jax.numpy as jnp
from jax import lax
from jax.experimental import pallas as pl
from jax.experimental.pallas import tpu as pltpu
from jax.experimental.pallas import tpu_sc as plsc

N = 10000
NPAD = 10112
RPT = 632
D = 256
DC = 128
E = 160000
NC = 2
NS = 16
B = 128
EPAD = 163840
NBTOT = EPAD // B
TILE_E = EPAD // (NC * NS)
NBATCH = TILE_E // B

CHUNK_T0 = [0, 2, 4, 7]
CHUNK_NS = [2, 2, 3, 3]


def _mm_body(x_ref, w_ref, o_ref):
    o_ref[0] = jnp.dot(x_ref[...], w_ref[0], preferred_element_type=jnp.float32)


def _make_tables(x, wstack):
    BN = 2000
    return pl.pallas_call(
        _mm_body,
        grid=(N // BN, 10),
        in_specs=[
            pl.BlockSpec((BN, D), lambda i, t: (i, 0)),
            pl.BlockSpec((1, D, DC), lambda i, t: (t, 0, 0)),
        ],
        out_specs=pl.BlockSpec((1, BN, DC), lambda i, t: (t, i, 0)),
        out_shape=jax.ShapeDtypeStruct((10, N, DC), jnp.float32),
    )(x, wstack)


def _sc_body(tabs, srcr, dstr, valr, zer,
             out, acc, sbuf, dbuf, vbuf, idx2, rows2, gsem, ssem):
    cid = lax.axis_index("c")
    sid = lax.axis_index("s")
    rowbase = cid * (NBTOT // NC) + sid * NBATCH

    for c in range(4):
        pltpu.sync_copy(zer, acc.at[pl.ds(sid * RPT, RPT), :])
        plsc.subcore_barrier()

        def _subtask(j, _, t0=CHUNK_T0[c]):
            s = j
            t = t0 + j
            pltpu.sync_copy(srcr.at[s, pl.ds(rowbase, NBATCH), :], sbuf)
            pltpu.sync_copy(dstr.at[s, pl.ds(rowbase, NBATCH), :], dbuf)
            pltpu.sync_copy(valr.at[s, pl.ds(rowbase, NBATCH), :], vbuf)

            def _prep(b, bi):
                for jj in range(8):
                    sl = pl.ds(jj * 16, 16)
                    idx2[bi, sl] = sbuf[b, sl] + t * N

            def _gather(bi):
                pltpu.async_copy(tabs.at[idx2.at[bi]], rows2.at[bi],
                                 gsem.at[bi])

            def _gwait(bi):
                pltpu.make_async_copy(tabs.at[idx2.at[bi]], rows2.at[bi],
                                      gsem.at[bi]).wait()

            def _swait(bi):
                pltpu.make_async_copy(rows2.at[bi], acc.at[dbuf.at[0]],
                                      ssem.at[bi]).wait()

            _prep(0, 0)
            _gather(0)

            def _batch(b, _):
                bi = b % 2
                bo = (b + 1) % 2

                @pl.when(b + 1 < NBATCH)
                def _():
                    @pl.when(b >= 1)
                    def _():
                        _swait(bo)
                    _prep(b + 1, bo)
                    _gather(bo)

                _gwait(bi)

                @plsc.parallel_loop(0, 8, unroll=2)
                def _scale(g):
                    vv = vbuf[b, pl.ds(g * 16, 16)]
                    for l in range(16):
                        v = vv[l]
                        e = g * 16 + l
                        loads = [rows2[bi, e, pl.ds(jj * 16, 16)]
                                 for jj in range(8)]
                        for jj in range(8):
                            rows2[bi, e, pl.ds(jj * 16, 16)] = loads[jj] * v

                pltpu.async_copy(rows2.at[bi], acc.at[dbuf.at[b]],
                                 ssem.at[bi], add=True)
                return 0

            lax.fori_loop(0, NBATCH, _batch, 0)
            _swait(0)
            _swait(1)
            return 0

        lax.fori_loop(0, CHUNK_NS[c], _subtask, 0)

        plsc.subcore_barrier()
        pltpu.sync_copy(acc.at[pl.ds(sid * RPT, RPT), :],
                        out.at[cid, c, pl.ds(sid * RPT, RPT), :])


def _sc_spmm(tabs, srcr, dstr, valr, zer):
    mesh = plsc.VectorSubcoreMesh(core_axis_name="c", subcore_axis_name="s")
    kern = functools.partial(
        pl.kernel,
        mesh=mesh,
        out_type=jax.ShapeDtypeStruct((NC, 4, NPAD, DC), jnp.float32),
        scratch_types=[
            pltpu.VMEM_SHARED((NPAD, DC), jnp.float32),
            pltpu.VMEM((NBATCH, B), jnp.int32),
            pltpu.VMEM((NBATCH, B), jnp.int32),
            pltpu.VMEM((NBATCH, B), jnp.float32),
            pltpu.VMEM((2, B), jnp.int32),
            pltpu.VMEM((2, B, DC), jnp.float32),
            pltpu.SemaphoreType.DMA((2,)),
            pltpu.SemaphoreType.DMA((2,)),
        ],
    )(_sc_body)
    return kern(tabs, srcr, dstr, valr, zer)


def _ep_body(p_ref, x_ref, o_ref):
    c0 = p_ref[0, 0] + p_ref[1, 0]
    c1 = p_ref[0, 1] + p_ref[1, 1]
    c2 = p_ref[0, 2] + p_ref[1, 2]
    c3 = p_ref[0, 3] + p_ref[1, 3]
    o_ref[:, 0:DC] = jnp.maximum(jnp.maximum(c0, c2), 0.0)
    o_ref[:, DC:2 * DC] = jnp.maximum(jnp.maximum(c1, c3), 0.0)
    o_ref[:, 2 * DC:] = x_ref[...]


def _epilogue(part, x):
    BN = 2000
    return pl.pallas_call(
        _ep_body,
        grid=(N // BN,),
        in_specs=[
            pl.BlockSpec((NC, 4, BN, DC), lambda i: (0, 0, i, 0)),
            pl.BlockSpec((BN, D), lambda i: (i, 0)),
        ],
        out_specs=pl.BlockSpec((BN, 2 * D), lambda i: (i, 0)),
        out_shape=jax.ShapeDtypeStruct((N, 2 * D), jnp.float32),
    )(part, x)


def kernel(x, s0_idx, s0_val, s1_idx, s1_val, s2_idx, s2_val,
           W00, W01, W10, W11, W12):
    x = x.astype(jnp.float32)
    wstack = jnp.stack([
        W00[:, :DC], W01[:, :DC],
        W00[:, DC:], W01[:, DC:],
        W10[:, :DC], W11[:, :DC], W12[:, :DC],
        W10[:, DC:], W11[:, DC:], W12[:, DC:],
    ])
    tabs = _make_tables(x, wstack).reshape(10 * N, DC)

    pad = EPAD - E
    srcs, dsts, vals = [], [], []
    for idx, val in ((s0_idx, s0_val), (s1_idx, s1_val), (s2_idx, s2_val)):
        idx = idx.astype(jnp.int32)
        srcs.append(jnp.pad(idx[1], (0, pad)).reshape(NBTOT, B))
        dsts.append(jnp.pad(idx[0], (0, pad)).reshape(NBTOT, B))
        vals.append(jnp.pad(val.astype(jnp.float32), (0, pad)).reshape(NBTOT, B))

    zer = jnp.zeros((RPT, DC), jnp.float32)
    part = _sc_spmm(tabs, jnp.stack(srcs), jnp.stack(dsts), jnp.stack(vals),
                    zer)
    return _epilogue(part, x)
